# baseline (speedup 1.0000x reference)
import jax
import jax.numpy as jnp
from jax import lax
from jax.experimental import pallas as pl
from jax.experimental.pallas import tpu as pltpu

_N = 16384
_C = 1000
_CW = 512
_NCH = _N // _CW
_NBUF = 4


def _body(y_ref, x_hbm, o_hbm, out_vm, sem, *bufs):
    def dma(g, b):
        return pltpu.make_async_copy(
            x_hbm.at[:, pl.ds(g * _CW, _CW)], bufs[b], sem.at[b]
        )

    for g in range(_NBUF):
        dma(g, g).start()

    for g in range(_NCH):
        b = g % _NBUF
        dma(g, b).wait()
        x = bufs[b][...]
        yv = y_ref[g, :]
        rows = lax.broadcasted_iota(jnp.int32, (_C, _CW), 0)
        sel = jnp.where(rows == yv[None, :], x, 0.0)
        out_vm[g, :] = jnp.sum(sel, axis=0)
        if g + _NBUF < _NCH:
            dma(g + _NBUF, b).start()

    cp = pltpu.make_async_copy(out_vm, o_hbm, sem.at[0])
    cp.start()
    cp.wait()


def kernel(logits, y):
    lt = logits.T
    y2 = y.astype(jnp.int32).reshape(_NCH, _CW)
    out = pl.pallas_call(
        _body,
        in_specs=[
            pl.BlockSpec(memory_space=pltpu.VMEM),
            pl.BlockSpec(memory_space=pl.ANY),
        ],
        out_specs=pl.BlockSpec(memory_space=pl.ANY),
        out_shape=jax.ShapeDtypeStruct((_NCH, _CW), jnp.float32),
        scratch_shapes=[
            pltpu.VMEM((_NCH, _CW), jnp.float32),
            pltpu.SemaphoreType.DMA((_NBUF,)),
        ]
        + [pltpu.VMEM((_C, _CW), jnp.float32) for _ in range(_NBUF)],
    )(y2, lt)
    return out.reshape(-1)


# speedup vs baseline: 1.0696x; 1.0696x over previous
import jax
import jax.numpy as jnp
from jax import lax
from jax.experimental import pallas as pl

_N = 16384
_C = 1000
_RB = 128
_NB = 8


def _body(y_ref, x_ref, o_ref):
    g = pl.program_id(0)
    y = y_ref[0, 0, :]
    x = x_ref[...]
    rows = lax.broadcasted_iota(jnp.int32, (_RB, _N), 0) + g * _RB
    sel = jnp.where(rows == y[None, :], x, 0.0)
    partial = jnp.sum(sel, axis=0)

    @pl.when(g == 0)
    def _init():
        o_ref[0, 0, :] = partial

    @pl.when(g > 0)
    def _acc():
        o_ref[0, 0, :] += partial


def kernel(logits, y):
    lt = logits.T
    y2 = y.astype(jnp.int32).reshape(1, 1, _N)
    out = pl.pallas_call(
        _body,
        grid=(_NB,),
        in_specs=[
            pl.BlockSpec((1, 1, _N), lambda i: (0, 0, 0)),
            pl.BlockSpec((_RB, _N), lambda i: (i, 0)),
        ],
        out_specs=pl.BlockSpec((1, 1, _N), lambda i: (0, 0, 0)),
        out_shape=jax.ShapeDtypeStruct((1, 1, _N), jnp.float32),
    )(y2, lt)
    return out.reshape(-1)
